# baseline (speedup 1.0000x reference)
import functools

import jax
import jax.numpy as jnp
from jax import lax
from jax.experimental import pallas as pl
from jax.experimental.pallas import tpu as pltpu
from jax.experimental.pallas import tpu_sc as plsc

B = 10000
NNN = 32
NAF = 128
NBF = 16
H1 = 128
H2 = 128
NCONV = 3
E = B * NNN

NSPLIT = 1
BC = B // NSPLIT
EC = E // NSPLIT

_NC = 2
_NS = 16
_NW = _NC * _NS
_BPW = EC // _NW
_CH = 400
_NCH = _BPW // _CH

_F32 = jnp.float32
_BF16 = jnp.bfloat16
_U32 = jnp.uint32
HP = H1 // 2


_NBUF = 2


@functools.cache
def _make_sc_gather(ec=EC, ch=_CH):
    mesh = plsc.VectorSubcoreMesh(
        core_axis_name="c", subcore_axis_name="s",
        num_cores=_NC, num_subcores=_NS,
    )
    _BPW = ec // _NW
    _NCH = _BPW // ch
    _CH = ch
    nring = _NCH // _NBUF * _NBUF
    _LEAD = 2

    @functools.partial(
        pl.kernel,
        out_type=jax.ShapeDtypeStruct((ec, H1), _F32),
        mesh=mesh,
        scratch_types=[
            pltpu.VMEM((_BPW,), jnp.int32),
            pltpu.VMEM((_NBUF, _CH, H1), _F32),
            [pltpu.SemaphoreType.DMA] * _NBUF,
            [pltpu.SemaphoreType.DMA] * _NBUF,
        ],
    )
    def sc_gather(table_hbm, idx_hbm, out_hbm, idx_v, rows_v, gsems, wsems):
        wid = lax.axis_index("s") * _NC + lax.axis_index("c")
        base = wid * _BPW
        pltpu.sync_copy(idx_hbm.at[pl.ds(base, _BPW)], idx_v)

        def g_copy(chunk, b):
            return pltpu.make_async_copy(
                table_hbm.at[idx_v.at[pl.ds(chunk * _CH, _CH)]],
                rows_v.at[b],
                gsems[b],
            )

        def w_copy(chunk, b):
            return pltpu.make_async_copy(
                rows_v.at[b],
                out_hbm.at[pl.ds(base + chunk * _CH, _CH)],
                wsems[b],
            )

        for c in range(min(_LEAD, _NCH)):
            g_copy(c, c % _NBUF).start()

        def step(cur, b):
            g_copy(cur, b).wait()
            w_copy(cur, b).start()
            pre = cur + _LEAD
            bp = (b + _LEAD) % _NBUF

            @pl.when(pre < _NCH)
            def _():
                @pl.when(pre >= _NBUF)
                def _():
                    w_copy(pre, bp).wait()

                g_copy(pre, bp).start()

        @pl.loop(0, nring, step=_NBUF)
        def _(k):
            for b in range(_NBUF):
                step(k + b, b)

        for cur in range(nring, _NCH):
            step(cur, cur % _NBUF)

        for j in range(max(0, _NCH - _NBUF), _NCH):
            w_copy(j, j % _NBUF).wait()

    return sc_gather


def _gather(g, idx_c):
    return _make_sc_gather()(g, idx_c).reshape(BC, NNN, H1)


def _softplus(x):
    return jnp.maximum(x, 0.0) + jnp.log1p(jnp.exp(-jnp.abs(x)))


def _dot(a, b):
    return jnp.dot(a, b, preferred_element_type=_F32)


_PM = 2000


def _pre_body(atom_ref, ws_ref, wn_ref, bias_ref, a_ref, g_ref):
    x = atom_ref[...]
    a_ref[...] = _dot(x, ws_ref[...]) + bias_ref[...]
    g_ref[...] = _dot(x, wn_ref[...])


def _pre(atom, w_self, w_nbr, bias2d):
    w_spec = pl.BlockSpec((NAF, H1), lambda i: (0, 0))
    return pl.pallas_call(
        _pre_body,
        grid=(B // _PM,),
        in_specs=[
            pl.BlockSpec((_PM, NAF), lambda i: (i, 0)),
            w_spec,
            w_spec,
            pl.BlockSpec((1, H1), lambda i: (0, 0)),
        ],
        out_specs=[
            pl.BlockSpec((_PM, H1), lambda i: (i, 0)),
            pl.BlockSpec((_PM, H1), lambda i: (i, 0)),
        ],
        out_shape=[
            jax.ShapeDtypeStruct((B, H1), _F32),
            jax.ShapeDtypeStruct((B, H1), _F32),
        ],
    )(atom, w_self, w_nbr, bias2d)


_BM = 400


def _round_body(bonds_ref, ng_ref, a_ref, ah_ref, wb_ref, wam_ref, waa_ref,
                ba_ref, ws_ref, wn_ref, bn_ref, wfc_ref, bfc_ref,
                bh_out, ah_out, a_out, g_out, y_out):
    kdim = bonds_ref.shape[2]
    x = bonds_ref[...].reshape(_BM * NNN, kdim)
    if x.dtype == jnp.int8:
        x = x.astype(_BF16)
    c = _dot(x, wb_ref[...]).reshape(_BM, NNN, H1)
    t = jnp.tanh(a_ref[...][:, None, :] + ng_ref[...] + c)
    bh_out[...] = jnp.round(t * 127.0).astype(jnp.int8)
    m = jnp.mean(t, axis=1)
    ah = jnp.maximum(
        _dot(m, wam_ref[...]) + _dot(ah_ref[...], waa_ref[...]) + ba_ref[...], 0.0
    )
    ah_out[...] = ah
    a_out[...] = _dot(ah, ws_ref[...]) + bn_ref[...]
    g_out[...] = _dot(ah, wn_ref[...])
    y_out[...] = _softplus(_dot(ah, wfc_ref[...]) + bfc_ref[...])[:, 0:1]


def _round(bonds_in, ng, a, ah, w_bond, w_am, w_aa, ba2, w_self, w_nbr, bn2,
           wfc_pad, bfc2, off):
    kdim = bonds_in.shape[2]
    ob = off * (BC // _BM)
    w128 = pl.BlockSpec((H1, H1), lambda i: (0, 0))
    b128 = pl.BlockSpec((1, H1), lambda i: (0, 0))
    row_in = pl.BlockSpec((_BM, H1), lambda i: (i + ob, 0))
    row_out = pl.BlockSpec((_BM, H1), lambda i: (i, 0))
    return pl.pallas_call(
        _round_body,
        grid=(BC // _BM,),
        in_specs=[
            pl.BlockSpec((_BM, NNN, kdim), lambda i: (i + ob, 0, 0)),
            pl.BlockSpec((_BM, NNN, H1), lambda i: (i, 0, 0)),
            row_in,
            row_in,
            pl.BlockSpec((kdim, H1), lambda i: (0, 0)),
            w128, w128, b128, w128, w128, b128,
            w128, b128,
        ],
        out_specs=[
            pl.BlockSpec((_BM, NNN, H1), lambda i: (i, 0, 0)),
            row_out, row_out, row_out,
            pl.BlockSpec((_BM, 1), lambda i: (i, 0)),
        ],
        out_shape=[
            jax.ShapeDtypeStruct((BC, NNN, H1), jnp.int8),
            jax.ShapeDtypeStruct((BC, H1), _F32),
            jax.ShapeDtypeStruct((BC, H1), _F32),
            jax.ShapeDtypeStruct((BC, H1), _F32),
            jax.ShapeDtypeStruct((BC, 1), _F32),
        ],
    )(bonds_in, ng, a, ah, w_bond, w_am, w_aa, ba2, w_self, w_nbr, bn2,
      wfc_pad, bfc2)


def kernel(gmap, atom, bonds, W_be, b_be, W_ae, b_ae, W_bu, b_bu, W_au, b_au,
           W_fc, b_fc):
    idx = gmap.astype(jnp.int32).reshape(E)
    idx_c = [lax.slice(idx, (c * EC,), ((c + 1) * EC,)) for c in range(NSPLIT)]

    wbe_s, wbe_n, wbe_b = W_be[:NAF], W_be[NAF:2 * NAF], W_be[2 * NAF:]
    wae_m, wae_a = W_ae[:H1], W_ae[H1:]
    wbu_s, wbu_n, wbu_b = W_bu[:H2], W_bu[H2:2 * H2], W_bu[2 * H2:]
    wau_m, wau_a = W_au[:H1], W_au[H1:]
    b_be2 = b_be.reshape(1, H1)
    b_ae2 = b_ae.reshape(1, H2)
    b_bu2 = b_bu.reshape(1, H1)
    b_au2 = b_au.reshape(1, H2)
    wbu_b16 = (wbu_b * (1.0 / 127.0)).astype(_BF16)
    wfc_pad = jnp.zeros((H2, H1), _F32).at[:, 0:1].set(W_fc)
    b_fc2 = jnp.broadcast_to(b_fc.reshape(1, 1), (1, H1))

    warm = _make_sc_gather(8192, 256)(atom, lax.slice(idx, (0,), (8192,)))

    a0, g0 = _pre(atom, wbe_s, wbe_n, b_be2)
    state = []
    for c in range(NSPLIT):
        ng = _gather(g0, idx_c[c])
        state.append(_round(
            bonds, ng, a0, atom, wbe_b, wae_m, wae_a, b_ae2, wbu_s, wbu_n,
            b_bu2, wfc_pad, b_fc2, off=c,
        ))
    g = jnp.concatenate([s[3] for s in state], axis=0)

    for _ in range(NCONV):
        new_state = []
        for c in range(NSPLIT):
            ng = _gather(g, idx_c[c])
            bh_c, ah_c, a_c = state[c][0], state[c][1], state[c][2]
            new_state.append(_round(
                bh_c, ng, a_c, ah_c, wbu_b16, wau_m, wau_a, b_au2, wbu_s,
                wbu_n, b_bu2, wfc_pad, b_fc2, off=0,
            ))
        state = new_state
        g = jnp.concatenate([s[3] for s in state], axis=0)

    y = jnp.concatenate([s[4] for s in state], axis=0)
    return y + warm[0:1, 0:1] * 1e-45


# speedup vs baseline: 1.0552x; 1.0552x over previous
import functools

import jax
import jax.numpy as jnp
from jax import lax
from jax.experimental import pallas as pl
from jax.experimental.pallas import tpu as pltpu
from jax.experimental.pallas import tpu_sc as plsc

B = 10000
NNN = 32
NAF = 128
NBF = 16
H1 = 128
H2 = 128
NCONV = 3
E = B * NNN

NSPLIT = 1
BC = B // NSPLIT
EC = E // NSPLIT

_NC = 2
_NS = 16
_NW = _NC * _NS
_BPW = EC // _NW
_CH = 400
_NCH = _BPW // _CH

_F32 = jnp.float32
_BF16 = jnp.bfloat16
_U32 = jnp.uint32
HP = H1 // 2


_NBUF = 2


@functools.cache
def _make_sc_gather(ec=EC, ch=_CH):
    mesh = plsc.VectorSubcoreMesh(
        core_axis_name="c", subcore_axis_name="s",
        num_cores=_NC, num_subcores=_NS,
    )
    _BPW = ec // _NW
    _NCH = _BPW // ch
    _CH = ch
    nring = _NCH // _NBUF * _NBUF
    _LEAD = 2

    @functools.partial(
        pl.kernel,
        out_type=jax.ShapeDtypeStruct((ec, H1), _F32),
        mesh=mesh,
        scratch_types=[
            pltpu.VMEM((_BPW,), jnp.int32),
            pltpu.VMEM((_NBUF, _CH, H1), _F32),
            [pltpu.SemaphoreType.DMA] * _NBUF,
            [pltpu.SemaphoreType.DMA] * _NBUF,
        ],
    )
    def sc_gather(table_hbm, idx_hbm, out_hbm, idx_v, rows_v, gsems, wsems):
        wid = lax.axis_index("s") * _NC + lax.axis_index("c")
        base = wid * _BPW
        pltpu.sync_copy(idx_hbm.at[pl.ds(base, _BPW)], idx_v)

        def g_copy(chunk, b):
            return pltpu.make_async_copy(
                table_hbm.at[idx_v.at[pl.ds(chunk * _CH, _CH)]],
                rows_v.at[b],
                gsems[b],
            )

        def w_copy(chunk, b):
            return pltpu.make_async_copy(
                rows_v.at[b],
                out_hbm.at[pl.ds(base + chunk * _CH, _CH)],
                wsems[b],
            )

        for c in range(min(_LEAD, _NCH)):
            g_copy(c, c % _NBUF).start()

        def step(cur, b):
            g_copy(cur, b).wait()
            w_copy(cur, b).start()
            pre = cur + _LEAD
            bp = (b + _LEAD) % _NBUF

            @pl.when(pre < _NCH)
            def _():
                @pl.when(pre >= _NBUF)
                def _():
                    w_copy(pre, bp).wait()

                g_copy(pre, bp).start()

        @pl.loop(0, nring, step=_NBUF)
        def _(k):
            for b in range(_NBUF):
                step(k + b, b)

        for cur in range(nring, _NCH):
            step(cur, cur % _NBUF)

        for j in range(max(0, _NCH - _NBUF), _NCH):
            w_copy(j, j % _NBUF).wait()

    return sc_gather


def _gather(g, idx_c):
    return _make_sc_gather()(g, idx_c).reshape(BC, NNN, H1)


def _softplus(x):
    return jnp.maximum(x, 0.0) + jnp.log1p(jnp.exp(-jnp.abs(x)))


def _dot(a, b):
    return jnp.dot(a, b, preferred_element_type=_F32)


_PM = 2000


def _pre_body(atom_ref, ws_ref, wn_ref, bias_ref, a_ref, g_ref):
    x = atom_ref[...]
    a_ref[...] = _dot(x, ws_ref[...]) + bias_ref[...]
    g_ref[...] = _dot(x, wn_ref[...])


def _pre(atom, w_self, w_nbr, bias2d):
    w_spec = pl.BlockSpec((NAF, H1), lambda i: (0, 0))
    return pl.pallas_call(
        _pre_body,
        grid=(B // _PM,),
        in_specs=[
            pl.BlockSpec((_PM, NAF), lambda i: (i, 0)),
            w_spec,
            w_spec,
            pl.BlockSpec((1, H1), lambda i: (0, 0)),
        ],
        out_specs=[
            pl.BlockSpec((_PM, H1), lambda i: (i, 0)),
            pl.BlockSpec((_PM, H1), lambda i: (i, 0)),
        ],
        out_shape=[
            jax.ShapeDtypeStruct((B, H1), _F32),
            jax.ShapeDtypeStruct((B, H1), _F32),
        ],
    )(atom, w_self, w_nbr, bias2d)


_BM = 400


def _round_core(bonds_ref, ng_ref, a_ref, ah_ref, wb_ref, wam_ref, waa_ref,
                ba_ref):
    kdim = bonds_ref.shape[2]
    x = bonds_ref[...].reshape(_BM * NNN, kdim)
    if x.dtype == jnp.int8:
        x = x.astype(_BF16)
    c = _dot(x, wb_ref[...]).reshape(_BM, NNN, H1)
    t = jnp.tanh(a_ref[...][:, None, :] + ng_ref[...] + c)
    m = jnp.mean(t, axis=1)
    ah = jnp.maximum(
        _dot(m, wam_ref[...]) + _dot(ah_ref[...], waa_ref[...]) + ba_ref[...], 0.0
    )
    return t, ah


def _round_body(bonds_ref, ng_ref, a_ref, ah_ref, wb_ref, wam_ref, waa_ref,
                ba_ref, ws_ref, wn_ref, bn_ref,
                bh_out, ah_out, a_out, g_out):
    t, ah = _round_core(bonds_ref, ng_ref, a_ref, ah_ref, wb_ref, wam_ref,
                        waa_ref, ba_ref)
    bh_out[...] = jnp.round(t * 127.0).astype(jnp.int8)
    ah_out[...] = ah
    a_out[...] = _dot(ah, ws_ref[...]) + bn_ref[...]
    g_out[...] = _dot(ah, wn_ref[...])


def _last_body(bonds_ref, ng_ref, a_ref, ah_ref, wb_ref, wam_ref, waa_ref,
               ba_ref, wfc_ref, bfc_ref, y_out):
    _, ah = _round_core(bonds_ref, ng_ref, a_ref, ah_ref, wb_ref, wam_ref,
                        waa_ref, ba_ref)
    y_out[...] = _softplus(_dot(ah, wfc_ref[...]) + bfc_ref[...])[:, 0:1]


def _common_specs(kdim, ob):
    w128 = pl.BlockSpec((H1, H1), lambda i: (0, 0))
    b128 = pl.BlockSpec((1, H1), lambda i: (0, 0))
    row_in = pl.BlockSpec((_BM, H1), lambda i: (i + ob, 0))
    return [
        pl.BlockSpec((_BM, NNN, kdim), lambda i: (i + ob, 0, 0)),
        pl.BlockSpec((_BM, NNN, H1), lambda i: (i, 0, 0)),
        row_in,
        row_in,
        pl.BlockSpec((kdim, H1), lambda i: (0, 0)),
        w128, w128, b128,
    ], w128, b128


def _round(bonds_in, ng, a, ah, w_bond, w_am, w_aa, ba2, w_self, w_nbr, bn2,
           off):
    kdim = bonds_in.shape[2]
    ob = off * (BC // _BM)
    specs, w128, b128 = _common_specs(kdim, ob)
    row_out = pl.BlockSpec((_BM, H1), lambda i: (i, 0))
    return pl.pallas_call(
        _round_body,
        grid=(BC // _BM,),
        in_specs=specs + [w128, w128, b128],
        out_specs=[
            pl.BlockSpec((_BM, NNN, H1), lambda i: (i, 0, 0)),
            row_out, row_out, row_out,
        ],
        out_shape=[
            jax.ShapeDtypeStruct((BC, NNN, H1), jnp.int8),
            jax.ShapeDtypeStruct((BC, H1), _F32),
            jax.ShapeDtypeStruct((BC, H1), _F32),
            jax.ShapeDtypeStruct((BC, H1), _F32),
        ],
    )(bonds_in, ng, a, ah, w_bond, w_am, w_aa, ba2, w_self, w_nbr, bn2)


def _last_round(bonds_in, ng, a, ah, w_bond, w_am, w_aa, ba2, wfc_pad, bfc2):
    kdim = bonds_in.shape[2]
    specs, w128, b128 = _common_specs(kdim, 0)
    return pl.pallas_call(
        _last_body,
        grid=(BC // _BM,),
        in_specs=specs + [w128, b128],
        out_specs=pl.BlockSpec((_BM, 1), lambda i: (i, 0)),
        out_shape=jax.ShapeDtypeStruct((BC, 1), _F32),
    )(bonds_in, ng, a, ah, w_bond, w_am, w_aa, ba2, wfc_pad, bfc2)


def kernel(gmap, atom, bonds, W_be, b_be, W_ae, b_ae, W_bu, b_bu, W_au, b_au,
           W_fc, b_fc):
    idx = gmap.astype(jnp.int32).reshape(E)
    idx_c = [lax.slice(idx, (c * EC,), ((c + 1) * EC,)) for c in range(NSPLIT)]

    wbe_s, wbe_n, wbe_b = W_be[:NAF], W_be[NAF:2 * NAF], W_be[2 * NAF:]
    wae_m, wae_a = W_ae[:H1], W_ae[H1:]
    wbu_s, wbu_n, wbu_b = W_bu[:H2], W_bu[H2:2 * H2], W_bu[2 * H2:]
    wau_m, wau_a = W_au[:H1], W_au[H1:]
    b_be2 = b_be.reshape(1, H1)
    b_ae2 = b_ae.reshape(1, H2)
    b_bu2 = b_bu.reshape(1, H1)
    b_au2 = b_au.reshape(1, H2)
    wbu_b16 = (wbu_b * (1.0 / 127.0)).astype(_BF16)
    wfc_pad = jnp.zeros((H2, H1), _F32).at[:, 0:1].set(W_fc)
    b_fc2 = jnp.broadcast_to(b_fc.reshape(1, 1), (1, H1))

    a0, g0 = _pre(atom, wbe_s, wbe_n, b_be2)
    ng = _gather(g0, idx_c[0])
    bh, ah, a, g = _round(bonds, ng, a0, atom, wbe_b, wae_m, wae_a, b_ae2,
                          wbu_s, wbu_n, b_bu2, off=0)

    for _ in range(NCONV - 1):
        ng = _gather(g, idx_c[0])
        bh, ah, a, g = _round(bh, ng, a, ah, wbu_b16, wau_m, wau_a, b_au2,
                              wbu_s, wbu_n, b_bu2, off=0)

    ng = _gather(g, idx_c[0])
    return _last_round(bh, ng, a, ah, wbu_b16, wau_m, wau_a, b_au2,
                       wfc_pad, b_fc2)
